# baseline (speedup 1.0000x reference)
import functools

import jax
import jax.numpy as jnp
from jax import lax
from jax.experimental import pallas as pl
from jax.experimental.pallas import tpu as pltpu
from jax.experimental.pallas import tpu_sc as plsc

N = 10000
E = 320000
D_IN = 128
D_HID = 256
D_OUT = 128
BLK = 100

NC = 2
NS = 16
NW = NC * NS
CHUNK = 128
G = 8
E_PAD = NW * CHUNK * 80
DEG_CHUNKS = E_PAD // (NW * CHUNK)
AGG_CHUNKS = E_PAD // (NS * CHUNK)
RPT = 632
NP = NS * RPT

_MESH = plsc.VectorSubcoreMesh(core_axis_name="c", subcore_axis_name="s")


def _agg_loop(g_hbm, src_hbm, dst_hbm, srow0, drow0, acc_sh,
              sidx_blk, didx_blk, rows_v, sem_i, sg, ss, ct):
    ngroups = ct // G
    dummy = g_hbm.at[pl.ds(0, CHUNK)]
    dummy_idx = src_hbm.at[pl.ds(0, G)]
    pltpu.sync_copy(src_hbm.at[pl.ds(srow0, G)], sidx_blk.at[0])
    pltpu.sync_copy(dst_hbm.at[pl.ds(drow0, G)], didx_blk.at[0])
    pltpu.async_copy(g_hbm.at[sidx_blk.at[0, 0]], rows_v.at[0], sg[0])

    def group(k, carry):
        s = k % 2
        not_last = k + 1 < ngroups
        for b in range(G):
            p = b % 2
            pltpu.make_async_copy(dummy, rows_v.at[p], sg[p]).wait()
            pltpu.async_copy(rows_v.at[p], acc_sh.at[didx_blk.at[s, b]],
                             ss[p], add=True)
            if b == 0:
                @pl.when(k > 0)
                def _wait_prev():
                    pltpu.make_async_copy(dummy, rows_v.at[1], ss[1]).wait()

                @pl.when(not_last)
                def _prefetch_idx():
                    pltpu.async_copy(src_hbm.at[pl.ds(srow0 + (k + 1) * G, G)],
                                     sidx_blk.at[1 - s], sem_i)
                    pltpu.async_copy(dst_hbm.at[pl.ds(drow0 + (k + 1) * G, G)],
                                     didx_blk.at[1 - s], sem_i)
            else:
                pltpu.make_async_copy(dummy, rows_v.at[1 - p], ss[1 - p]).wait()
            if b < G - 1:
                pltpu.async_copy(g_hbm.at[sidx_blk.at[s, b + 1]],
                                 rows_v.at[1 - p], sg[1 - p])
            else:
                @pl.when(not_last)
                def _fire_next():
                    pltpu.make_async_copy(dummy_idx, sidx_blk.at[0], sem_i).wait()
                    pltpu.make_async_copy(dummy_idx, didx_blk.at[0], sem_i).wait()
                    pltpu.async_copy(g_hbm.at[sidx_blk.at[1 - s, 0]],
                                     rows_v.at[1 - p], sg[1 - p])
        return carry

    lax.fori_loop(0, ngroups, group, 0)
    pltpu.make_async_copy(dummy, rows_v.at[1], ss[1]).wait()


@functools.partial(
    pl.kernel,
    out_type=jax.ShapeDtypeStruct((2 * NP, 128), jnp.float32),
    mesh=_MESH,
    scratch_types=[
        pltpu.VMEM((DEG_CHUNKS, CHUNK), jnp.int32),
        pltpu.VMEM((CHUNK, 128), jnp.float32),
        pltpu.VMEM_SHARED((NP, 128), jnp.float32),
        pltpu.SemaphoreType.DMA,
    ],
)
def _deg_call(dst_hbm, zeros_hbm, ones_hbm, out_hbm, didx_all, ones_v, acc_sh,
              sem):
    cid = lax.axis_index("c")
    sid = lax.axis_index("s")
    wid = sid * NC + cid
    pltpu.sync_copy(dst_hbm.at[pl.ds(wid * DEG_CHUNKS, DEG_CHUNKS)], didx_all)
    pltpu.sync_copy(zeros_hbm.at[pl.ds(sid * RPT, RPT)],
                    acc_sh.at[pl.ds(sid * RPT, RPT)])
    pltpu.sync_copy(ones_hbm, ones_v)
    plsc.subcore_barrier()

    def group(k, carry):
        for b in range(8):
            pltpu.async_copy(ones_v, acc_sh.at[didx_all.at[k * 8 + b]], sem,
                             add=True)
        for b in range(8):
            pltpu.make_async_copy(ones_hbm, ones_v, sem).wait()
        return carry

    lax.fori_loop(0, DEG_CHUNKS // 8, group, 0)
    plsc.subcore_barrier()
    pltpu.sync_copy(acc_sh.at[pl.ds(sid * RPT, RPT)],
                    out_hbm.at[pl.ds(cid * NP + sid * RPT, RPT)])


@functools.partial(
    pl.kernel,
    out_type=jax.ShapeDtypeStruct((2 * NP, 128), jnp.float32),
    mesh=_MESH,
    scratch_types=[
        pltpu.VMEM((2, G, CHUNK), jnp.int32),
        pltpu.VMEM((2, G, CHUNK), jnp.int32),
        pltpu.VMEM((2, CHUNK, 128), jnp.float32),
        pltpu.VMEM_SHARED((NP, 128), jnp.float32),
        pltpu.SemaphoreType.DMA,
        pltpu.SemaphoreType.DMA,
        pltpu.SemaphoreType.DMA,
        pltpu.SemaphoreType.DMA,
        pltpu.SemaphoreType.DMA,
    ],
)
def _agg128(g_hbm, src2_hbm, dst_hbm, out_hbm, sidx_blk, didx_blk, rows_v,
            acc_sh, si, s0, s1, t0, t1):
    cid = lax.axis_index("c")
    sid = lax.axis_index("s")
    pltpu.sync_copy(g_hbm.at[pl.ds(cid * NP + sid * RPT, RPT)],
                    acc_sh.at[pl.ds(sid * RPT, RPT)])
    plsc.subcore_barrier()
    _agg_loop(g_hbm, src2_hbm, dst_hbm,
              cid * (E_PAD // CHUNK) + sid * AGG_CHUNKS, sid * AGG_CHUNKS,
              acc_sh, sidx_blk, didx_blk, rows_v, si, (s0, s1), (t0, t1),
              AGG_CHUNKS)
    plsc.subcore_barrier()
    pltpu.sync_copy(acc_sh.at[pl.ds(sid * RPT, RPT)],
                    out_hbm.at[pl.ds(cid * NP + sid * RPT, RPT)])


EDGE_HALF = E_PAD // NC
EDGE_CHUNKS = E_PAD // (NW * CHUNK)


@functools.partial(
    pl.kernel,
    out_type=jax.ShapeDtypeStruct((2 * NP, 128), jnp.float32),
    mesh=_MESH,
    scratch_types=[
        pltpu.VMEM((2, G, CHUNK), jnp.int32),
        pltpu.VMEM((2, G, CHUNK), jnp.int32),
        pltpu.VMEM((2, CHUNK, 128), jnp.float32),
        pltpu.VMEM_SHARED((NP, 128), jnp.float32),
        pltpu.SemaphoreType.DMA,
        pltpu.SemaphoreType.DMA,
        pltpu.SemaphoreType.DMA,
        pltpu.SemaphoreType.DMA,
        pltpu.SemaphoreType.DMA,
    ],
)
def _agg_edge(g_hbm, gh_hbm, src_hbm, dst_hbm, out_hbm,
              sidx_blk, didx_blk, rows_v, acc_sh, si, s0, s1, t0, t1):
    cid = lax.axis_index("c")
    sid = lax.axis_index("s")
    row0 = cid * (EDGE_HALF // CHUNK) + sid * EDGE_CHUNKS
    pltpu.sync_copy(gh_hbm.at[pl.ds(sid * RPT, RPT)],
                    acc_sh.at[pl.ds(sid * RPT, RPT)])
    plsc.subcore_barrier()
    _agg_loop(g_hbm, src_hbm, dst_hbm, row0, row0,
              acc_sh, sidx_blk, didx_blk, rows_v, si, (s0, s1), (t0, t1),
              EDGE_CHUNKS)
    plsc.subcore_barrier()
    pltpu.sync_copy(acc_sh.at[pl.ds(sid * RPT, RPT)],
                    out_hbm.at[pl.ds(cid * NP + sid * RPT, RPT)])


def _b_body(x_ref, w1_ref, b1_ref, p0_ref, p1_ref, g1_ref, dinv_ref):
    deg = 1.0 + p0_ref[0, :, :1] + p1_ref[0, :, :1]
    dinv = lax.rsqrt(deg)
    h = jnp.dot(x_ref[...], w1_ref[...], preferred_element_type=jnp.float32)
    g = dinv * (h + b1_ref[...])
    g1_ref[0] = g[:, :128]
    g1_ref[1] = g[:, 128:]
    dinv_ref[...] = dinv


def _b_call(x, w1, b1, degp):
    return pl.pallas_call(
        _b_body,
        grid=(NS,),
        in_specs=[
            pl.BlockSpec((RPT, D_IN), lambda i: (i, 0)),
            pl.BlockSpec((D_IN, D_HID), lambda i: (0, 0)),
            pl.BlockSpec((1, D_HID), lambda i: (0, 0)),
            pl.BlockSpec((1, RPT, 128), lambda i: (0, i, 0)),
            pl.BlockSpec((1, RPT, 128), lambda i: (1, i, 0)),
        ],
        out_specs=[
            pl.BlockSpec((2, RPT, 128), lambda i: (0, i, 0)),
            pl.BlockSpec((RPT, 1), lambda i: (i, 0)),
        ],
        out_shape=[
            jax.ShapeDtypeStruct((2, NP, 128), jnp.float32),
            jax.ShapeDtypeStruct((NP, 1), jnp.float32),
        ],
    )(x, w1, b1, degp, degp)


def _d_body(agg_ref, dinv_ref, w2a_ref, w2b_ref, b2_ref, g2_ref, gh_ref):
    dinv = dinv_ref[...]
    h1a = jnp.tanh(dinv * agg_ref[0])
    h1b = jnp.tanh(dinv * agg_ref[1])
    h = jnp.dot(h1a, w2a_ref[...], preferred_element_type=jnp.float32)
    h = h + jnp.dot(h1b, w2b_ref[...], preferred_element_type=jnp.float32)
    g2 = dinv * (h + b2_ref[...])
    g2_ref[...] = g2
    gh_ref[...] = 0.5 * g2


def _d_call(agg1, dinv, w2a, w2b, b2):
    return pl.pallas_call(
        _d_body,
        grid=(NS,),
        in_specs=[
            pl.BlockSpec((2, RPT, 128), lambda i: (0, i, 0)),
            pl.BlockSpec((RPT, 1), lambda i: (i, 0)),
            pl.BlockSpec((128, D_OUT), lambda i: (0, 0)),
            pl.BlockSpec((128, D_OUT), lambda i: (0, 0)),
            pl.BlockSpec((1, D_OUT), lambda i: (0, 0)),
        ],
        out_specs=[
            pl.BlockSpec((RPT, D_OUT), lambda i: (i, 0)),
            pl.BlockSpec((RPT, D_OUT), lambda i: (i, 0)),
        ],
        out_shape=[
            jax.ShapeDtypeStruct((NP, D_OUT), jnp.float32),
            jax.ShapeDtypeStruct((NP, D_OUT), jnp.float32),
        ],
    )(agg1, dinv, w2a, w2b, b2)


def _f_body(agg_ref, dinv_ref, out_ref):
    dinv = dinv_ref[0]
    h2 = jnp.tanh(dinv * (agg_ref[0, 0] + agg_ref[1, 0]))
    dn = (((1,), (1,)), ((), ()))
    p = lax.dot_general(h2, h2, dn, preferred_element_type=jnp.float32)
    row = lax.broadcasted_iota(jnp.int32, (BLK, BLK), 0)
    col = lax.broadcasted_iota(jnp.int32, (BLK, BLK), 1)
    out_ref[0] = jnp.where(row == col, 0.0, p)


def _f_call(agg2, dinv):
    return pl.pallas_call(
        _f_body,
        grid=(N // BLK,),
        in_specs=[
            pl.BlockSpec((2, 1, BLK, 128), lambda i: (0, i, 0, 0)),
            pl.BlockSpec((1, BLK, 1), lambda i: (i, 0, 0)),
        ],
        out_specs=pl.BlockSpec((1, BLK, BLK), lambda i: (i, 0, 0)),
        out_shape=jax.ShapeDtypeStruct((N // BLK, BLK, BLK), jnp.float32),
    )(agg2, dinv)


def kernel(x, edge_index, eyes, W1, b1, W2, b2):
    src = edge_index[0].astype(jnp.int32)
    dst = edge_index[1].astype(jnp.int32)
    pad = E_PAD - E
    srcp = jnp.concatenate([src, jnp.arange(pad, dtype=jnp.int32) % N])
    junk = N + (jnp.arange(pad, dtype=jnp.int32) % (NP - N))
    dstp = jnp.concatenate([dst, junk])
    src2 = jnp.concatenate([srcp, srcp + NP]).reshape(-1, CHUNK)
    srcp = srcp.reshape(-1, CHUNK)
    dstp = dstp.reshape(-1, CHUNK)

    xp = jnp.pad(x, ((0, NP - N), (0, 0)))
    zeros_init = jnp.zeros((NP, 128), jnp.float32)
    ones_c = jnp.ones((CHUNK, 128), jnp.float32)

    degp = _deg_call(dstp, zeros_init, ones_c).reshape(2, NP, 128)
    g1, dinv = _b_call(xp, W1, b1.reshape(1, -1), degp)
    agg1 = _agg128(g1.reshape(2 * NP, 128), src2, dstp)
    g2, g2h = _d_call(agg1.reshape(2, NP, 128), dinv, W2[:128], W2[128:],
                      b2.reshape(1, -1))
    agg2 = _agg_edge(g2, g2h, srcp, dstp)
    out = _f_call(agg2.reshape(2, NP, 128)[:, :N].reshape(2, N // BLK, BLK, 128),
                  dinv[:N].reshape(N // BLK, BLK, 1))
    return out.reshape(N, BLK)


# speedup vs baseline: 1.1448x; 1.1448x over previous
import functools

import jax
import jax.numpy as jnp
from jax import lax
from jax.experimental import pallas as pl
from jax.experimental.pallas import tpu as pltpu
from jax.experimental.pallas import tpu_sc as plsc

N = 10000
E = 320000
D_IN = 128
D_HID = 256
D_OUT = 128
BLK = 100

NC = 2
NS = 16
NW = NC * NS
CHUNK = 128
G = 8
E_PAD = NW * CHUNK * 80
DEG_CHUNKS = E_PAD // (NW * CHUNK)
AGG_CHUNKS = E_PAD // (NS * CHUNK)
RPT = 632
NP = NS * RPT

_MESH = plsc.VectorSubcoreMesh(core_axis_name="c", subcore_axis_name="s")


def _agg_loop(g_hbm, src_hbm, dst_hbm, srow0, drow0, acc_sh,
              sidx_blk, didx_blk, rows_v, sem_i, sg, ss, ct):
    ngroups = ct // G
    dummy = g_hbm.at[pl.ds(0, CHUNK)]
    dummy_idx = src_hbm.at[pl.ds(0, G)]
    pltpu.sync_copy(src_hbm.at[pl.ds(srow0, G)], sidx_blk.at[0])
    pltpu.sync_copy(dst_hbm.at[pl.ds(drow0, G)], didx_blk.at[0])
    pltpu.async_copy(g_hbm.at[sidx_blk.at[0, 0]], rows_v.at[0], sg[0])
    pltpu.async_copy(g_hbm.at[sidx_blk.at[0, 1]], rows_v.at[1], sg[1])

    def group(k, carry):
        s = k % 2
        not_last = k + 1 < ngroups

        @pl.when(not_last)
        def _prefetch_idx():
            pltpu.async_copy(src_hbm.at[pl.ds(srow0 + (k + 1) * G, G)],
                             sidx_blk.at[1 - s], sem_i)
            pltpu.async_copy(dst_hbm.at[pl.ds(drow0 + (k + 1) * G, G)],
                             didx_blk.at[1 - s], sem_i)

        for b in range(G):
            p = b % 2
            pltpu.make_async_copy(dummy, rows_v.at[p], sg[p]).wait()
            pltpu.sync_copy(rows_v.at[p], acc_sh.at[didx_blk.at[s, b]],
                            add=True)
            if b == G - 2:
                @pl.when(not_last)
                def _wait_idx():
                    pltpu.make_async_copy(dummy_idx, sidx_blk.at[0], sem_i).wait()
                    pltpu.make_async_copy(dummy_idx, didx_blk.at[0], sem_i).wait()
            if b < G - 2:
                pltpu.async_copy(g_hbm.at[sidx_blk.at[s, b + 2]],
                                 rows_v.at[p], sg[p])
            else:
                @pl.when(not_last)
                def _fire_next():
                    pltpu.async_copy(g_hbm.at[sidx_blk.at[1 - s, b - (G - 2)]],
                                     rows_v.at[p], sg[p])
        return carry

    lax.fori_loop(0, ngroups, group, 0)


@functools.partial(
    pl.kernel,
    out_type=jax.ShapeDtypeStruct((2 * NP, 128), jnp.float32),
    mesh=_MESH,
    scratch_types=[
        pltpu.VMEM((DEG_CHUNKS, CHUNK), jnp.int32),
        pltpu.VMEM((CHUNK, 128), jnp.float32),
        pltpu.VMEM_SHARED((NP, 128), jnp.float32),
        pltpu.SemaphoreType.DMA,
    ],
)
def _deg_call(dst_hbm, zeros_hbm, ones_hbm, out_hbm, didx_all, ones_v, acc_sh,
              sem):
    cid = lax.axis_index("c")
    sid = lax.axis_index("s")
    wid = sid * NC + cid
    pltpu.sync_copy(dst_hbm.at[pl.ds(wid * DEG_CHUNKS, DEG_CHUNKS)], didx_all)
    pltpu.sync_copy(zeros_hbm.at[pl.ds(sid * RPT, RPT)],
                    acc_sh.at[pl.ds(sid * RPT, RPT)])
    pltpu.sync_copy(ones_hbm, ones_v)
    plsc.subcore_barrier()

    def group(k, carry):
        for b in range(8):
            pltpu.async_copy(ones_v, acc_sh.at[didx_all.at[k * 8 + b]], sem,
                             add=True)
        for b in range(8):
            pltpu.make_async_copy(ones_hbm, ones_v, sem).wait()
        return carry

    lax.fori_loop(0, DEG_CHUNKS // 8, group, 0)
    plsc.subcore_barrier()
    pltpu.sync_copy(acc_sh.at[pl.ds(sid * RPT, RPT)],
                    out_hbm.at[pl.ds(cid * NP + sid * RPT, RPT)])


@functools.partial(
    pl.kernel,
    out_type=jax.ShapeDtypeStruct((2 * NP, 128), jnp.float32),
    mesh=_MESH,
    scratch_types=[
        pltpu.VMEM((2, G, CHUNK), jnp.int32),
        pltpu.VMEM((2, G, CHUNK), jnp.int32),
        pltpu.VMEM((2, CHUNK, 128), jnp.float32),
        pltpu.VMEM_SHARED((NP, 128), jnp.float32),
        pltpu.SemaphoreType.DMA,
        pltpu.SemaphoreType.DMA,
        pltpu.SemaphoreType.DMA,
        pltpu.SemaphoreType.DMA,
        pltpu.SemaphoreType.DMA,
    ],
)
def _agg128(g_hbm, src2_hbm, dst_hbm, out_hbm, sidx_blk, didx_blk, rows_v,
            acc_sh, si, s0, s1, t0, t1):
    cid = lax.axis_index("c")
    sid = lax.axis_index("s")
    pltpu.sync_copy(g_hbm.at[pl.ds(cid * NP + sid * RPT, RPT)],
                    acc_sh.at[pl.ds(sid * RPT, RPT)])
    plsc.subcore_barrier()
    _agg_loop(g_hbm, src2_hbm, dst_hbm,
              cid * (E_PAD // CHUNK) + sid * AGG_CHUNKS, sid * AGG_CHUNKS,
              acc_sh, sidx_blk, didx_blk, rows_v, si, (s0, s1), (t0, t1),
              AGG_CHUNKS)
    plsc.subcore_barrier()
    pltpu.sync_copy(acc_sh.at[pl.ds(sid * RPT, RPT)],
                    out_hbm.at[pl.ds(cid * NP + sid * RPT, RPT)])


EDGE_HALF = E_PAD // NC
EDGE_CHUNKS = E_PAD // (NW * CHUNK)


@functools.partial(
    pl.kernel,
    out_type=jax.ShapeDtypeStruct((2 * NP, 128), jnp.float32),
    mesh=_MESH,
    scratch_types=[
        pltpu.VMEM((2, G, CHUNK), jnp.int32),
        pltpu.VMEM((2, G, CHUNK), jnp.int32),
        pltpu.VMEM((2, CHUNK, 128), jnp.float32),
        pltpu.VMEM_SHARED((NP, 128), jnp.float32),
        pltpu.SemaphoreType.DMA,
        pltpu.SemaphoreType.DMA,
        pltpu.SemaphoreType.DMA,
        pltpu.SemaphoreType.DMA,
        pltpu.SemaphoreType.DMA,
    ],
)
def _agg_edge(g_hbm, gh_hbm, src_hbm, dst_hbm, out_hbm,
              sidx_blk, didx_blk, rows_v, acc_sh, si, s0, s1, t0, t1):
    cid = lax.axis_index("c")
    sid = lax.axis_index("s")
    row0 = cid * (EDGE_HALF // CHUNK) + sid * EDGE_CHUNKS
    pltpu.sync_copy(gh_hbm.at[pl.ds(sid * RPT, RPT)],
                    acc_sh.at[pl.ds(sid * RPT, RPT)])
    plsc.subcore_barrier()
    _agg_loop(g_hbm, src_hbm, dst_hbm, row0, row0,
              acc_sh, sidx_blk, didx_blk, rows_v, si, (s0, s1), (t0, t1),
              EDGE_CHUNKS)
    plsc.subcore_barrier()
    pltpu.sync_copy(acc_sh.at[pl.ds(sid * RPT, RPT)],
                    out_hbm.at[pl.ds(cid * NP + sid * RPT, RPT)])


def _b_body(x_ref, w1_ref, b1_ref, p0_ref, p1_ref, g1_ref, dinv_ref):
    deg = 1.0 + p0_ref[0, :, :1] + p1_ref[0, :, :1]
    dinv = lax.rsqrt(deg)
    h = jnp.dot(x_ref[...], w1_ref[...], preferred_element_type=jnp.float32)
    g = dinv * (h + b1_ref[...])
    g1_ref[0] = g[:, :128]
    g1_ref[1] = g[:, 128:]
    dinv_ref[...] = dinv


_RB = 1000


def _b_call(x, w1, b1, degp):
    return pl.pallas_call(
        _b_body,
        grid=(N // _RB,),
        in_specs=[
            pl.BlockSpec((_RB, D_IN), lambda i: (i, 0)),
            pl.BlockSpec((D_IN, D_HID), lambda i: (0, 0)),
            pl.BlockSpec((1, D_HID), lambda i: (0, 0)),
            pl.BlockSpec((1, _RB, 128), lambda i: (0, i, 0)),
            pl.BlockSpec((1, _RB, 128), lambda i: (1, i, 0)),
        ],
        out_specs=[
            pl.BlockSpec((2, _RB, 128), lambda i: (0, i, 0)),
            pl.BlockSpec((_RB, 1), lambda i: (i, 0)),
        ],
        out_shape=[
            jax.ShapeDtypeStruct((2, NP, 128), jnp.float32),
            jax.ShapeDtypeStruct((NP, 1), jnp.float32),
        ],
    )(x, w1, b1, degp, degp)


def _d_body(agg_ref, dinv_ref, w2a_ref, w2b_ref, b2_ref, g2_ref, gh_ref):
    dinv = dinv_ref[...]
    h1a = jnp.tanh(dinv * agg_ref[0])
    h1b = jnp.tanh(dinv * agg_ref[1])
    h = jnp.dot(h1a, w2a_ref[...], preferred_element_type=jnp.float32)
    h = h + jnp.dot(h1b, w2b_ref[...], preferred_element_type=jnp.float32)
    g2 = dinv * (h + b2_ref[...])
    g2_ref[...] = g2
    gh_ref[...] = 0.5 * g2


def _d_call(agg1, dinv, w2a, w2b, b2):
    return pl.pallas_call(
        _d_body,
        grid=(N // _RB,),
        in_specs=[
            pl.BlockSpec((2, _RB, 128), lambda i: (0, i, 0)),
            pl.BlockSpec((_RB, 1), lambda i: (i, 0)),
            pl.BlockSpec((128, D_OUT), lambda i: (0, 0)),
            pl.BlockSpec((128, D_OUT), lambda i: (0, 0)),
            pl.BlockSpec((1, D_OUT), lambda i: (0, 0)),
        ],
        out_specs=[
            pl.BlockSpec((_RB, D_OUT), lambda i: (i, 0)),
            pl.BlockSpec((_RB, D_OUT), lambda i: (i, 0)),
        ],
        out_shape=[
            jax.ShapeDtypeStruct((NP, D_OUT), jnp.float32),
            jax.ShapeDtypeStruct((NP, D_OUT), jnp.float32),
        ],
    )(agg1, dinv, w2a, w2b, b2)


def _f_body(agg_ref, dinv_ref, out_ref):
    dinv = dinv_ref[0]
    h2 = jnp.tanh(dinv * (agg_ref[0, 0] + agg_ref[1, 0]))
    dn = (((1,), (1,)), ((), ()))
    p = lax.dot_general(h2, h2, dn, preferred_element_type=jnp.float32)
    row = lax.broadcasted_iota(jnp.int32, (BLK, BLK), 0)
    col = lax.broadcasted_iota(jnp.int32, (BLK, BLK), 1)
    out_ref[0] = jnp.where(row == col, 0.0, p)


def _f_call(agg2, dinv):
    return pl.pallas_call(
        _f_body,
        grid=(N // BLK,),
        in_specs=[
            pl.BlockSpec((2, 1, BLK, 128), lambda i: (0, i, 0, 0)),
            pl.BlockSpec((1, BLK, 1), lambda i: (i, 0, 0)),
        ],
        out_specs=pl.BlockSpec((1, BLK, BLK), lambda i: (i, 0, 0)),
        out_shape=jax.ShapeDtypeStruct((N // BLK, BLK, BLK), jnp.float32),
    )(agg2, dinv)


def kernel(x, edge_index, eyes, W1, b1, W2, b2):
    src = edge_index[0].astype(jnp.int32)
    dst = edge_index[1].astype(jnp.int32)
    pad = E_PAD - E
    srcp = jnp.concatenate([src, jnp.arange(pad, dtype=jnp.int32) % N])
    junk = N + (jnp.arange(pad, dtype=jnp.int32) % (NP - N))
    dstp = jnp.concatenate([dst, junk])
    src2 = jnp.concatenate([srcp, srcp + NP]).reshape(-1, CHUNK)
    srcp = srcp.reshape(-1, CHUNK)
    dstp = dstp.reshape(-1, CHUNK)

    zeros_init = jnp.zeros((NP, 128), jnp.float32)
    ones_c = jnp.ones((CHUNK, 128), jnp.float32)

    degp = _deg_call(dstp, zeros_init, ones_c).reshape(2, NP, 128)
    g1, dinv = _b_call(x, W1, b1.reshape(1, -1), degp)
    agg1 = _agg128(g1.reshape(2 * NP, 128), src2, dstp)
    g2, g2h = _d_call(agg1.reshape(2, NP, 128), dinv, W2[:128], W2[128:],
                      b2.reshape(1, -1))
    agg2 = _agg_edge(g2, g2h, srcp, dstp)
    out = _f_call(agg2.reshape(2, NP, 128)[:, :N].reshape(2, N // BLK, BLK, 128),
                  dinv[:N].reshape(N // BLK, BLK, 1))
    return out.reshape(N, BLK)


# speedup vs baseline: 1.1830x; 1.0334x over previous
import functools

import jax
import jax.numpy as jnp
from jax import lax
from jax.experimental import pallas as pl
from jax.experimental.pallas import tpu as pltpu
from jax.experimental.pallas import tpu_sc as plsc

N = 10000
E = 320000
D_IN = 128
D_HID = 256
D_OUT = 128
BLK = 100

NC = 2
NS = 16
NW = NC * NS
CHUNK = 128
G = 8
E_PAD = NW * CHUNK * 80
DEG_CHUNKS = E_PAD // (NW * CHUNK)
AGG_CHUNKS = E_PAD // (NS * CHUNK)
RPT = 632
NP = NS * RPT

_MESH = plsc.VectorSubcoreMesh(core_axis_name="c", subcore_axis_name="s")


def _agg_loop(g_hbm, src_hbm, dst_hbm, srow0, drow0, acc_sh,
              sidx_blk, didx_blk, rows_v, sem_i, sg, ct):
    ngroups = ct // G
    dummy = g_hbm.at[pl.ds(0, CHUNK)]
    dummy_idx = src_hbm.at[pl.ds(0, G)]
    pltpu.sync_copy(src_hbm.at[pl.ds(srow0, G)], sidx_blk.at[0])
    pltpu.sync_copy(dst_hbm.at[pl.ds(drow0, G)], didx_blk.at[0])
    pltpu.async_copy(g_hbm.at[sidx_blk.at[0, 0]], rows_v.at[0], sg[0])
    pltpu.async_copy(g_hbm.at[sidx_blk.at[0, 1]], rows_v.at[1], sg[1])

    def group(k, carry):
        s = k % 2
        not_last = k + 1 < ngroups

        @pl.when(not_last)
        def _prefetch_idx():
            pltpu.async_copy(src_hbm.at[pl.ds(srow0 + (k + 1) * G, G)],
                             sidx_blk.at[1 - s], sem_i)
            pltpu.async_copy(dst_hbm.at[pl.ds(drow0 + (k + 1) * G, G)],
                             didx_blk.at[1 - s], sem_i)

        for b in range(G):
            p = b % 2
            pltpu.make_async_copy(dummy, rows_v.at[p], sg[p]).wait()
            pltpu.sync_copy(rows_v.at[p], acc_sh.at[didx_blk.at[s, b]],
                            add=True)
            if b == G - 2:
                @pl.when(not_last)
                def _wait_idx():
                    pltpu.make_async_copy(dummy_idx, sidx_blk.at[0], sem_i).wait()
                    pltpu.make_async_copy(dummy_idx, didx_blk.at[0], sem_i).wait()
            if b < G - 2:
                pltpu.async_copy(g_hbm.at[sidx_blk.at[s, b + 2]],
                                 rows_v.at[p], sg[p])
            else:
                @pl.when(not_last)
                def _fire_next():
                    pltpu.async_copy(g_hbm.at[sidx_blk.at[1 - s, b - (G - 2)]],
                                     rows_v.at[p], sg[p])
        return carry

    lax.fori_loop(0, ngroups, group, 0)


ND = NP
RPT_D = ND // NS


@functools.partial(
    pl.kernel,
    out_type=jax.ShapeDtypeStruct((2 * ND, 128), jnp.float32),
    mesh=_MESH,
    scratch_types=[
        pltpu.VMEM((DEG_CHUNKS, CHUNK), jnp.int32),
        pltpu.VMEM((CHUNK, 128), jnp.float32),
        pltpu.VMEM_SHARED((ND, 128), jnp.float32),
        pltpu.SemaphoreType.DMA,
    ],
)
def _deg_call(dst_hbm, zeros_hbm, ones_hbm, out_hbm, didx_all, ones_v, acc_sh,
              sem):
    cid = lax.axis_index("c")
    sid = lax.axis_index("s")
    wid = sid * NC + cid
    pltpu.sync_copy(dst_hbm.at[pl.ds(wid * DEG_CHUNKS, DEG_CHUNKS)], didx_all)
    pltpu.sync_copy(zeros_hbm.at[pl.ds(sid * RPT_D, RPT_D)],
                    acc_sh.at[pl.ds(sid * RPT_D, RPT_D)])
    pltpu.sync_copy(ones_hbm, ones_v)
    plsc.subcore_barrier()

    def group(k, carry):
        for b in range(8):
            pltpu.async_copy(ones_v, acc_sh.at[didx_all.at[k * 8 + b]], sem,
                             add=True)
        for b in range(8):
            pltpu.make_async_copy(ones_hbm, ones_v, sem).wait()
        return carry

    lax.fori_loop(0, DEG_CHUNKS // 8, group, 0)
    plsc.subcore_barrier()
    pltpu.sync_copy(acc_sh.at[pl.ds(sid * RPT_D, RPT_D)],
                    out_hbm.at[pl.ds(cid * ND + sid * RPT_D, RPT_D)])


C1 = 96
G1 = 24
A1_CHUNKS = 216
E_PAD1 = NS * C1 * A1_CHUNKS


def _agg_loop3(g_hbm, src_hbm, dst_hbm, srow0, drow0, acc_sh,
               sidx_blk, didx_blk, rows_v, sem_i, sg, ss, ct):
    ngroups = ct // G1
    dummy = g_hbm.at[pl.ds(0, C1)]
    dummy_idx = src_hbm.at[pl.ds(0, G1)]
    pltpu.sync_copy(src_hbm.at[pl.ds(srow0, G1)], sidx_blk.at[0])
    pltpu.sync_copy(dst_hbm.at[pl.ds(drow0, G1)], didx_blk.at[0])
    pltpu.async_copy(g_hbm.at[sidx_blk.at[0, 0]], rows_v.at[0], sg[0])
    pltpu.async_copy(g_hbm.at[sidx_blk.at[0, 1]], rows_v.at[1], sg[1])

    def group(k, carry):
        s = k % 2
        not_last = k + 1 < ngroups
        for b in range(G1):
            p = b % 3
            pn = (b + 2) % 3
            pltpu.make_async_copy(dummy, rows_v.at[p], sg[p]).wait()
            pltpu.async_copy(rows_v.at[p], acc_sh.at[didx_blk.at[s, b]],
                             ss[p], add=True)
            if b == 0:
                @pl.when(k > 0)
                def _wait_prev():
                    pltpu.make_async_copy(dummy, rows_v.at[pn], ss[pn]).wait()

                @pl.when(not_last)
                def _prefetch_idx():
                    pltpu.async_copy(src_hbm.at[pl.ds(srow0 + (k + 1) * G1, G1)],
                                     sidx_blk.at[1 - s], sem_i)
                    pltpu.async_copy(dst_hbm.at[pl.ds(drow0 + (k + 1) * G1, G1)],
                                     didx_blk.at[1 - s], sem_i)
            else:
                pltpu.make_async_copy(dummy, rows_v.at[pn], ss[pn]).wait()
            if b < G1 - 2:
                pltpu.async_copy(g_hbm.at[sidx_blk.at[s, b + 2]],
                                 rows_v.at[pn], sg[pn])
            elif b == G1 - 2:
                @pl.when(not_last)
                def _fire_b4():
                    pltpu.make_async_copy(dummy_idx, sidx_blk.at[0], sem_i).wait()
                    pltpu.make_async_copy(dummy_idx, didx_blk.at[0], sem_i).wait()
                    pltpu.async_copy(g_hbm.at[sidx_blk.at[1 - s, 0]],
                                     rows_v.at[pn], sg[pn])
            else:
                @pl.when(not_last)
                def _fire_b5():
                    pltpu.async_copy(g_hbm.at[sidx_blk.at[1 - s, 1]],
                                     rows_v.at[pn], sg[pn])
        return carry

    lax.fori_loop(0, ngroups, group, 0)
    pltpu.make_async_copy(dummy, rows_v.at[(ct - 1) % 3], ss[(ct - 1) % 3]).wait()


@functools.partial(
    pl.kernel,
    out_type=jax.ShapeDtypeStruct((2 * NP, 128), jnp.float32),
    mesh=_MESH,
    scratch_types=[
        pltpu.VMEM((2, G1, C1), jnp.int32),
        pltpu.VMEM((2, G1, C1), jnp.int32),
        pltpu.VMEM((3, C1, 128), jnp.float32),
        pltpu.VMEM_SHARED((NP, 128), jnp.float32),
        pltpu.SemaphoreType.DMA,
        pltpu.SemaphoreType.DMA,
        pltpu.SemaphoreType.DMA,
        pltpu.SemaphoreType.DMA,
        pltpu.SemaphoreType.DMA,
        pltpu.SemaphoreType.DMA,
        pltpu.SemaphoreType.DMA,
    ],
)
def _agg128(g_hbm, src2_hbm, dst_hbm, out_hbm, sidx_blk, didx_blk, rows_v,
            acc_sh, si, s0, s1, s2, t0, t1, t2):
    cid = lax.axis_index("c")
    sid = lax.axis_index("s")
    pltpu.sync_copy(g_hbm.at[pl.ds(cid * NP + sid * RPT, RPT)],
                    acc_sh.at[pl.ds(sid * RPT, RPT)])
    plsc.subcore_barrier()
    _agg_loop3(g_hbm, src2_hbm, dst_hbm,
               cid * (E_PAD1 // C1) + sid * A1_CHUNKS, sid * A1_CHUNKS,
               acc_sh, sidx_blk, didx_blk, rows_v, si, (s0, s1, s2),
               (t0, t1, t2), A1_CHUNKS)
    plsc.subcore_barrier()
    pltpu.sync_copy(acc_sh.at[pl.ds(sid * RPT, RPT)],
                    out_hbm.at[pl.ds(cid * NP + sid * RPT, RPT)])


EDGE_HALF = E_PAD // NC
EDGE_CHUNKS = E_PAD // (NW * CHUNK)


@functools.partial(
    pl.kernel,
    out_type=jax.ShapeDtypeStruct((2 * NP, 128), jnp.float32),
    mesh=_MESH,
    scratch_types=[
        pltpu.VMEM((2, G, CHUNK), jnp.int32),
        pltpu.VMEM((2, G, CHUNK), jnp.int32),
        pltpu.VMEM((2, CHUNK, 128), jnp.float32),
        pltpu.VMEM_SHARED((NP, 128), jnp.float32),
        pltpu.SemaphoreType.DMA,
        pltpu.SemaphoreType.DMA,
        pltpu.SemaphoreType.DMA,
    ],
)
def _agg_edge(g_hbm, gh_hbm, src_hbm, dst_hbm, out_hbm,
              sidx_blk, didx_blk, rows_v, acc_sh, si, s0, s1):
    cid = lax.axis_index("c")
    sid = lax.axis_index("s")
    row0 = cid * (EDGE_HALF // CHUNK) + sid * EDGE_CHUNKS
    pltpu.sync_copy(gh_hbm.at[pl.ds(sid * RPT, RPT)],
                    acc_sh.at[pl.ds(sid * RPT, RPT)])
    plsc.subcore_barrier()
    _agg_loop(g_hbm, src_hbm, dst_hbm, row0, row0,
              acc_sh, sidx_blk, didx_blk, rows_v, si, (s0, s1), EDGE_CHUNKS)
    plsc.subcore_barrier()
    pltpu.sync_copy(acc_sh.at[pl.ds(sid * RPT, RPT)],
                    out_hbm.at[pl.ds(cid * NP + sid * RPT, RPT)])


def _b_body(x_ref, w1_ref, b1_ref, p0_ref, p1_ref, g1_ref, dinv_ref):
    deg = (1.0 + p0_ref[0, :, :1].astype(jnp.float32)
           + p1_ref[0, :, :1].astype(jnp.float32))
    dinv = lax.rsqrt(deg)
    h = jnp.dot(x_ref[...], w1_ref[...], preferred_element_type=jnp.float32)
    g = dinv * (h + b1_ref[...])
    g1_ref[0] = g[:, :128]
    g1_ref[1] = g[:, 128:]
    dinv_ref[...] = dinv


_RB = 2000


def _b_call(x, w1, b1, degp):
    return pl.pallas_call(
        _b_body,
        grid=(N // _RB,),
        in_specs=[
            pl.BlockSpec((_RB, D_IN), lambda i: (i, 0)),
            pl.BlockSpec((D_IN, D_HID), lambda i: (0, 0)),
            pl.BlockSpec((1, D_HID), lambda i: (0, 0)),
            pl.BlockSpec((1, _RB, 128), lambda i: (0, i, 0)),
            pl.BlockSpec((1, _RB, 128), lambda i: (1, i, 0)),
        ],
        out_specs=[
            pl.BlockSpec((2, _RB, 128), lambda i: (0, i, 0)),
            pl.BlockSpec((_RB, 1), lambda i: (i, 0)),
        ],
        out_shape=[
            jax.ShapeDtypeStruct((2, NP, 128), jnp.float32),
            jax.ShapeDtypeStruct((NP, 1), jnp.float32),
        ],
    )(x, w1, b1, degp, degp)


def _d_body(agg_ref, dinv_ref, w2a_ref, w2b_ref, b2_ref, g2_ref, gh_ref):
    dinv = dinv_ref[...]
    h1a = jnp.tanh(dinv * agg_ref[0])
    h1b = jnp.tanh(dinv * agg_ref[1])
    h = jnp.dot(h1a, w2a_ref[...], preferred_element_type=jnp.float32)
    h = h + jnp.dot(h1b, w2b_ref[...], preferred_element_type=jnp.float32)
    g2 = dinv * (h + b2_ref[...])
    g2_ref[...] = g2
    gh_ref[...] = 0.5 * g2


def _d_call(agg1, dinv, w2a, w2b, b2):
    return pl.pallas_call(
        _d_body,
        grid=(N // _RB,),
        in_specs=[
            pl.BlockSpec((2, _RB, 128), lambda i: (0, i, 0)),
            pl.BlockSpec((_RB, 1), lambda i: (i, 0)),
            pl.BlockSpec((128, D_OUT), lambda i: (0, 0)),
            pl.BlockSpec((128, D_OUT), lambda i: (0, 0)),
            pl.BlockSpec((1, D_OUT), lambda i: (0, 0)),
        ],
        out_specs=[
            pl.BlockSpec((_RB, D_OUT), lambda i: (i, 0)),
            pl.BlockSpec((_RB, D_OUT), lambda i: (i, 0)),
        ],
        out_shape=[
            jax.ShapeDtypeStruct((NP, D_OUT), jnp.float32),
            jax.ShapeDtypeStruct((NP, D_OUT), jnp.float32),
        ],
    )(agg1, dinv, w2a, w2b, b2)


def _f_body(agg_ref, dinv_ref, out_ref):
    dinv = dinv_ref[0]
    h2 = jnp.tanh(dinv * (agg_ref[0, 0] + agg_ref[1, 0]))
    dn = (((1,), (1,)), ((), ()))
    p = lax.dot_general(h2, h2, dn, preferred_element_type=jnp.float32)
    row = lax.broadcasted_iota(jnp.int32, (BLK, BLK), 0)
    col = lax.broadcasted_iota(jnp.int32, (BLK, BLK), 1)
    out_ref[0] = jnp.where(row == col, 0.0, p)


def _f_call(agg2, dinv):
    return pl.pallas_call(
        _f_body,
        grid=(N // BLK,),
        in_specs=[
            pl.BlockSpec((2, 1, BLK, 128), lambda i: (0, i, 0, 0)),
            pl.BlockSpec((1, BLK, 1), lambda i: (i, 0, 0)),
        ],
        out_specs=pl.BlockSpec((1, BLK, BLK), lambda i: (i, 0, 0)),
        out_shape=jax.ShapeDtypeStruct((N // BLK, BLK, BLK), jnp.float32),
    )(agg2, dinv)


def kernel(x, edge_index, eyes, W1, b1, W2, b2):
    src = edge_index[0].astype(jnp.int32)
    dst = edge_index[1].astype(jnp.int32)
    pad = E_PAD - E
    srcp = jnp.concatenate([src, jnp.arange(pad, dtype=jnp.int32) % N])
    junk = N + (jnp.arange(pad, dtype=jnp.int32) % (NP - N))
    dstp = jnp.concatenate([dst, junk])
    src2 = jnp.concatenate([srcp, srcp + NP]).reshape(-1, CHUNK)
    srcp = srcp.reshape(-1, CHUNK)
    dstp = dstp.reshape(-1, CHUNK)

    pad1 = E_PAD1 - E
    srcp1 = jnp.concatenate([src, jnp.arange(pad1, dtype=jnp.int32) % N])
    junk1 = N + (jnp.arange(pad1, dtype=jnp.int32) % (NP - N))
    dstp1 = jnp.concatenate([dst, junk1]).reshape(-1, C1)
    src2_1 = jnp.concatenate([srcp1, srcp1 + NP]).reshape(-1, C1)

    zeros_init = jnp.zeros((ND, 128), jnp.float32)
    ones_c = jnp.ones((CHUNK, 128), jnp.float32)

    degp = _deg_call(dstp, zeros_init, ones_c).reshape(2, ND, 128)
    g1, dinv = _b_call(x, W1, b1.reshape(1, -1), degp)
    agg1 = _agg128(g1.reshape(2 * NP, 128), src2_1, dstp1)
    g2, g2h = _d_call(agg1.reshape(2, NP, 128), dinv, W2[:128], W2[128:],
                      b2.reshape(1, -1))
    agg2 = _agg_edge(g2, g2h, srcp, dstp)
    out = _f_call(agg2.reshape(2, NP, 128)[:, :N].reshape(2, N // BLK, BLK, 128),
                  dinv[:N].reshape(N // BLK, BLK, 1))
    return out.reshape(N, BLK)


# speedup vs baseline: 1.3122x; 1.1092x over previous
import functools

import jax
import jax.numpy as jnp
from jax import lax
from jax.experimental import pallas as pl
from jax.experimental.pallas import tpu as pltpu
from jax.experimental.pallas import tpu_sc as plsc

N = 10000
E = 320000
D_IN = 128
D_HID = 256
D_OUT = 128
BLK = 100

NC = 2
NS = 16
NW = NC * NS
CHUNK = 128
G = 8
E_PAD = NW * CHUNK * 80
DEG_CHUNKS = E_PAD // (NW * CHUNK)
AGG_CHUNKS = E_PAD // (NS * CHUNK)
RPT = 632
NP = NS * RPT

_MESH = plsc.VectorSubcoreMesh(core_axis_name="c", subcore_axis_name="s")


def _agg_loop(g_hbm, src_hbm, dst_hbm, srow0, drow0, acc_sh,
              sidx_blk, didx_blk, rows_v, sem_i, sg, ct):
    ngroups = ct // G
    dummy = g_hbm.at[pl.ds(0, CHUNK)]
    dummy_idx = src_hbm.at[pl.ds(0, G)]
    pltpu.sync_copy(src_hbm.at[pl.ds(srow0, G)], sidx_blk.at[0])
    pltpu.sync_copy(dst_hbm.at[pl.ds(drow0, G)], didx_blk.at[0])
    pltpu.async_copy(g_hbm.at[sidx_blk.at[0, 0]], rows_v.at[0], sg[0])
    pltpu.async_copy(g_hbm.at[sidx_blk.at[0, 1]], rows_v.at[1], sg[1])

    def group(k, carry):
        s = k % 2
        not_last = k + 1 < ngroups

        @pl.when(not_last)
        def _prefetch_idx():
            pltpu.async_copy(src_hbm.at[pl.ds(srow0 + (k + 1) * G, G)],
                             sidx_blk.at[1 - s], sem_i)
            pltpu.async_copy(dst_hbm.at[pl.ds(drow0 + (k + 1) * G, G)],
                             didx_blk.at[1 - s], sem_i)

        for b in range(G):
            p = b % 2
            pltpu.make_async_copy(dummy, rows_v.at[p], sg[p]).wait()
            pltpu.sync_copy(rows_v.at[p], acc_sh.at[didx_blk.at[s, b]],
                            add=True)
            if b == G - 2:
                @pl.when(not_last)
                def _wait_idx():
                    pltpu.make_async_copy(dummy_idx, sidx_blk.at[0], sem_i).wait()
                    pltpu.make_async_copy(dummy_idx, didx_blk.at[0], sem_i).wait()
            if b < G - 2:
                pltpu.async_copy(g_hbm.at[sidx_blk.at[s, b + 2]],
                                 rows_v.at[p], sg[p])
            else:
                @pl.when(not_last)
                def _fire_next():
                    pltpu.async_copy(g_hbm.at[sidx_blk.at[1 - s, b - (G - 2)]],
                                     rows_v.at[p], sg[p])
        return carry

    lax.fori_loop(0, ngroups, group, 0)


ND = NP
RPT_D = ND // NS


@functools.partial(
    pl.kernel,
    out_type=jax.ShapeDtypeStruct((2 * ND, 128), jnp.float32),
    mesh=_MESH,
    scratch_types=[
        pltpu.VMEM((DEG_CHUNKS, CHUNK), jnp.int32),
        pltpu.VMEM((CHUNK, 128), jnp.float32),
        pltpu.VMEM_SHARED((ND, 128), jnp.float32),
        pltpu.SemaphoreType.DMA,
    ],
)
def _deg_call(dst_hbm, zeros_hbm, ones_hbm, out_hbm, didx_all, ones_v, acc_sh,
              sem):
    cid = lax.axis_index("c")
    sid = lax.axis_index("s")
    wid = sid * NC + cid
    pltpu.sync_copy(dst_hbm.at[pl.ds(wid * DEG_CHUNKS, DEG_CHUNKS)], didx_all)
    pltpu.sync_copy(zeros_hbm.at[pl.ds(sid * RPT_D, RPT_D)],
                    acc_sh.at[pl.ds(sid * RPT_D, RPT_D)])
    pltpu.sync_copy(ones_hbm, ones_v)
    plsc.subcore_barrier()

    def group(k, carry):
        for b in range(8):
            pltpu.async_copy(ones_v, acc_sh.at[didx_all.at[k * 8 + b]], sem,
                             add=True)
        for b in range(8):
            pltpu.make_async_copy(ones_hbm, ones_v, sem).wait()
        return carry

    lax.fori_loop(0, DEG_CHUNKS // 8, group, 0)
    plsc.subcore_barrier()
    pltpu.sync_copy(acc_sh.at[pl.ds(sid * RPT_D, RPT_D)],
                    out_hbm.at[pl.ds(cid * ND + sid * RPT_D, RPT_D)])


C1 = 96
G1 = 24
A1_CHUNKS = 216
E_PAD1 = NS * C1 * A1_CHUNKS


def _agg_loop3(g_hbm, src_hbm, dst_hbm, srow0, drow0, acc_sh,
               sidx_blk, didx_blk, rows_v, sem_i, sg, ss, ct):
    ngroups = ct // G1
    dummy = g_hbm.at[pl.ds(0, C1)]
    dummy_idx = src_hbm.at[pl.ds(0, G1)]
    pltpu.sync_copy(src_hbm.at[pl.ds(srow0, G1)], sidx_blk.at[0])
    pltpu.sync_copy(dst_hbm.at[pl.ds(drow0, G1)], didx_blk.at[0])
    pltpu.async_copy(g_hbm.at[sidx_blk.at[0, 0]], rows_v.at[0], sg[0])
    pltpu.async_copy(g_hbm.at[sidx_blk.at[0, 1]], rows_v.at[1], sg[1])

    def group(k, carry):
        s = k % 2
        not_last = k + 1 < ngroups
        for b in range(G1):
            p = b % 3
            pn = (b + 2) % 3
            pltpu.make_async_copy(dummy, rows_v.at[p], sg[p]).wait()
            pltpu.async_copy(rows_v.at[p], acc_sh.at[didx_blk.at[s, b]],
                             ss[p], add=True)
            if b == 0:
                @pl.when(k > 0)
                def _wait_prev():
                    pltpu.make_async_copy(dummy, rows_v.at[pn], ss[pn]).wait()

                @pl.when(not_last)
                def _prefetch_idx():
                    pltpu.async_copy(src_hbm.at[pl.ds(srow0 + (k + 1) * G1, G1)],
                                     sidx_blk.at[1 - s], sem_i)
                    pltpu.async_copy(dst_hbm.at[pl.ds(drow0 + (k + 1) * G1, G1)],
                                     didx_blk.at[1 - s], sem_i)
            else:
                pltpu.make_async_copy(dummy, rows_v.at[pn], ss[pn]).wait()
            if b < G1 - 2:
                pltpu.async_copy(g_hbm.at[sidx_blk.at[s, b + 2]],
                                 rows_v.at[pn], sg[pn])
            elif b == G1 - 2:
                @pl.when(not_last)
                def _fire_b4():
                    pltpu.make_async_copy(dummy_idx, sidx_blk.at[0], sem_i).wait()
                    pltpu.make_async_copy(dummy_idx, didx_blk.at[0], sem_i).wait()
                    pltpu.async_copy(g_hbm.at[sidx_blk.at[1 - s, 0]],
                                     rows_v.at[pn], sg[pn])
            else:
                @pl.when(not_last)
                def _fire_b5():
                    pltpu.async_copy(g_hbm.at[sidx_blk.at[1 - s, 1]],
                                     rows_v.at[pn], sg[pn])
        return carry

    lax.fori_loop(0, ngroups, group, 0)
    pltpu.make_async_copy(dummy, rows_v.at[(ct - 1) % 3], ss[(ct - 1) % 3]).wait()


@functools.partial(
    pl.kernel,
    out_type=jax.ShapeDtypeStruct((2 * NP, 128), jnp.float32),
    mesh=_MESH,
    scratch_types=[
        pltpu.VMEM((2, G1, C1), jnp.int32),
        pltpu.VMEM((2, G1, C1), jnp.int32),
        pltpu.VMEM((3, C1, 128), jnp.float32),
        pltpu.VMEM_SHARED((NP, 128), jnp.float32),
        pltpu.SemaphoreType.DMA,
        pltpu.SemaphoreType.DMA,
        pltpu.SemaphoreType.DMA,
        pltpu.SemaphoreType.DMA,
        pltpu.SemaphoreType.DMA,
        pltpu.SemaphoreType.DMA,
        pltpu.SemaphoreType.DMA,
    ],
)
def _agg128(g_hbm, src2_hbm, dst_hbm, out_hbm, sidx_blk, didx_blk, rows_v,
            acc_sh, si, s0, s1, s2, t0, t1, t2):
    cid = lax.axis_index("c")
    sid = lax.axis_index("s")
    pltpu.sync_copy(g_hbm.at[pl.ds(cid * NP + sid * RPT, RPT)],
                    acc_sh.at[pl.ds(sid * RPT, RPT)])
    plsc.subcore_barrier()
    _agg_loop3(g_hbm, src2_hbm, dst_hbm,
               cid * (E_PAD1 // C1) + sid * A1_CHUNKS, sid * A1_CHUNKS,
               acc_sh, sidx_blk, didx_blk, rows_v, si, (s0, s1, s2),
               (t0, t1, t2), A1_CHUNKS)
    plsc.subcore_barrier()
    pltpu.sync_copy(acc_sh.at[pl.ds(sid * RPT, RPT)],
                    out_hbm.at[pl.ds(cid * NP + sid * RPT, RPT)])


EDGE_HALF = E_PAD // NC
EDGE_CHUNKS = E_PAD // (NW * CHUNK)


@functools.partial(
    pl.kernel,
    out_type=jax.ShapeDtypeStruct((2 * NP, 128), jnp.float32),
    mesh=_MESH,
    scratch_types=[
        pltpu.VMEM((2, G, CHUNK), jnp.int32),
        pltpu.VMEM((2, G, CHUNK), jnp.int32),
        pltpu.VMEM((2, CHUNK, 128), jnp.float32),
        pltpu.VMEM_SHARED((NP, 128), jnp.float32),
        pltpu.SemaphoreType.DMA,
        pltpu.SemaphoreType.DMA,
        pltpu.SemaphoreType.DMA,
    ],
)
def _agg_edge(g_hbm, gh_hbm, src_hbm, dst_hbm, out_hbm,
              sidx_blk, didx_blk, rows_v, acc_sh, si, s0, s1):
    cid = lax.axis_index("c")
    sid = lax.axis_index("s")
    row0 = cid * (EDGE_HALF // CHUNK) + sid * EDGE_CHUNKS
    pltpu.sync_copy(gh_hbm.at[pl.ds(sid * RPT, RPT)],
                    acc_sh.at[pl.ds(sid * RPT, RPT)])
    plsc.subcore_barrier()
    _agg_loop(g_hbm, src_hbm, dst_hbm, row0, row0,
              acc_sh, sidx_blk, didx_blk, rows_v, si, (s0, s1), EDGE_CHUNKS)
    plsc.subcore_barrier()
    pltpu.sync_copy(acc_sh.at[pl.ds(sid * RPT, RPT)],
                    out_hbm.at[pl.ds(cid * NP + sid * RPT, RPT)])


def _b_body(x_ref, w1_ref, b1_ref, p0_ref, p1_ref, g1_ref, dinv_ref):
    deg = (1.0 + p0_ref[0, :, :1].astype(jnp.float32)
           + p1_ref[0, :, :1].astype(jnp.float32))
    dinv = lax.rsqrt(deg)
    h = jnp.dot(x_ref[...], w1_ref[...], preferred_element_type=jnp.float32)
    g = dinv * (h + b1_ref[...])
    g1_ref[0] = g[:, :128]
    g1_ref[1] = g[:, 128:]
    dinv_ref[...] = dinv


_RB = 2000


def _b_call(x, w1, b1, degp):
    return pl.pallas_call(
        _b_body,
        grid=(N // _RB,),
        in_specs=[
            pl.BlockSpec((_RB, D_IN), lambda i: (i, 0)),
            pl.BlockSpec((D_IN, D_HID), lambda i: (0, 0)),
            pl.BlockSpec((1, D_HID), lambda i: (0, 0)),
            pl.BlockSpec((1, _RB, 128), lambda i: (0, i, 0)),
            pl.BlockSpec((1, _RB, 128), lambda i: (1, i, 0)),
        ],
        out_specs=[
            pl.BlockSpec((2, _RB, 128), lambda i: (0, i, 0)),
            pl.BlockSpec((_RB, 1), lambda i: (i, 0)),
        ],
        out_shape=[
            jax.ShapeDtypeStruct((2, NP, 128), jnp.float32),
            jax.ShapeDtypeStruct((NP, 1), jnp.float32),
        ],
    )(x, w1, b1, degp, degp)


def _d_body(agg_ref, dinv_ref, w2a_ref, w2b_ref, b2_ref, g2_ref, gh_ref):
    dinv = dinv_ref[...]
    h1a = jnp.tanh(dinv * agg_ref[0])
    h1b = jnp.tanh(dinv * agg_ref[1])
    h = jnp.dot(h1a, w2a_ref[...], preferred_element_type=jnp.float32)
    h = h + jnp.dot(h1b, w2b_ref[...], preferred_element_type=jnp.float32)
    g2 = dinv * (h + b2_ref[...])
    g2_ref[...] = g2
    gh_ref[...] = 0.5 * g2


def _d_call(agg1, dinv, w2a, w2b, b2):
    return pl.pallas_call(
        _d_body,
        grid=(N // _RB,),
        in_specs=[
            pl.BlockSpec((2, _RB, 128), lambda i: (0, i, 0)),
            pl.BlockSpec((_RB, 1), lambda i: (i, 0)),
            pl.BlockSpec((128, D_OUT), lambda i: (0, 0)),
            pl.BlockSpec((128, D_OUT), lambda i: (0, 0)),
            pl.BlockSpec((1, D_OUT), lambda i: (0, 0)),
        ],
        out_specs=[
            pl.BlockSpec((_RB, D_OUT), lambda i: (i, 0)),
            pl.BlockSpec((_RB, D_OUT), lambda i: (i, 0)),
        ],
        out_shape=[
            jax.ShapeDtypeStruct((NP, D_OUT), jnp.float32),
            jax.ShapeDtypeStruct((NP, D_OUT), jnp.float32),
        ],
    )(agg1, dinv, w2a, w2b, b2)


_NB = 10


def _f_body(agg_ref, dinv_ref, out_ref):
    dn = (((1,), (1,)), ((), ()))
    row = lax.broadcasted_iota(jnp.int32, (BLK, BLK), 0)
    col = lax.broadcasted_iota(jnp.int32, (BLK, BLK), 1)
    for t in range(_NB):
        h2 = jnp.tanh(dinv_ref[t] * (agg_ref[0, t] + agg_ref[1, t]))
        p = lax.dot_general(h2, h2, dn, preferred_element_type=jnp.float32)
        out_ref[t] = jnp.where(row == col, 0.0, p)


def _f_call(agg2, dinv):
    return pl.pallas_call(
        _f_body,
        grid=(N // BLK // _NB,),
        in_specs=[
            pl.BlockSpec((2, _NB, BLK, 128), lambda i: (0, i, 0, 0)),
            pl.BlockSpec((_NB, BLK, 1), lambda i: (i, 0, 0)),
        ],
        out_specs=pl.BlockSpec((_NB, BLK, BLK), lambda i: (i, 0, 0)),
        out_shape=jax.ShapeDtypeStruct((N // BLK, BLK, BLK), jnp.float32),
    )(agg2, dinv)


def kernel(x, edge_index, eyes, W1, b1, W2, b2):
    src = edge_index[0].astype(jnp.int32)
    dst = edge_index[1].astype(jnp.int32)
    pad = E_PAD - E
    srcp = jnp.concatenate([src, jnp.arange(pad, dtype=jnp.int32) % N])
    junk = N + (jnp.arange(pad, dtype=jnp.int32) % (NP - N))
    dstp = jnp.concatenate([dst, junk])
    src2 = jnp.concatenate([srcp, srcp + NP]).reshape(-1, CHUNK)
    srcp = srcp.reshape(-1, CHUNK)
    dstp = dstp.reshape(-1, CHUNK)

    pad1 = E_PAD1 - E
    srcp1 = jnp.concatenate([src, jnp.arange(pad1, dtype=jnp.int32) % N])
    junk1 = N + (jnp.arange(pad1, dtype=jnp.int32) % (NP - N))
    dstp1 = jnp.concatenate([dst, junk1]).reshape(-1, C1)
    src2_1 = jnp.concatenate([srcp1, srcp1 + NP]).reshape(-1, C1)

    zeros_init = jnp.zeros((ND, 128), jnp.float32)
    ones_c = jnp.ones((CHUNK, 128), jnp.float32)

    degp = _deg_call(dstp, zeros_init, ones_c).reshape(2, ND, 128)
    g1, dinv = _b_call(x, W1, b1.reshape(1, -1), degp)
    agg1 = _agg128(g1.reshape(2 * NP, 128), src2_1, dstp1)
    g2, g2h = _d_call(agg1.reshape(2, NP, 128), dinv, W2[:128], W2[128:],
                      b2.reshape(1, -1))
    agg2 = _agg_edge(g2, g2h, srcp, dstp)
    out = _f_call(agg2.reshape(2, NP, 128)[:, :N].reshape(2, N // BLK, BLK, 128),
                  dinv[:N].reshape(N // BLK, BLK, 1))
    return out.reshape(N, BLK)
